# Initial kernel scaffold; baseline (speedup 1.0000x reference)
import jax
import jax.numpy as jnp
from jax.experimental import pallas as pl
from jax.experimental.pallas import tpu as pltpu
from jax.experimental.pallas import tpu_sc as plsc

B, L = 4096, 50
N = B * L
WORD_DIM = 64
POS_DIM = 32
REL_DIM = 32
W = 128
GRID = N // W


def _sc_gather3(word_idxs, pos_idxs, rel_idxs, word_table, pos_table, rel_table):
    mesh = plsc.VectorSubcoreMesh(core_axis_name="c", subcore_axis_name="s")

    @pl.kernel(
        out_type=(
            jax.ShapeDtypeStruct((N, WORD_DIM), jnp.float32),
            jax.ShapeDtypeStruct((N, POS_DIM), jnp.float32),
            jax.ShapeDtypeStruct((N, REL_DIM), jnp.float32),
        ),
        mesh=mesh,
    )
    def kern(wt_hbm, pt_hbm, rt_hbm, wi_hbm, pi_hbm, ri_hbm,
             wo_hbm, po_hbm, ro_hbm):
        def body(wi_v, pi_v, ri_v, wo_v, po_v, ro_v):
            pltpu.sync_copy(wt_hbm.at[wi_v.at[0]], wo_v)
            pltpu.sync_copy(pt_hbm.at[pi_v.at[0]], po_v)
            pltpu.sync_copy(rt_hbm.at[ri_v.at[0]], ro_v)

        pltpu.emit_pipeline(
            body,
            grid=(GRID,),
            in_specs=[
                pl.BlockSpec((1, W), lambda i: (0, i)),
                pl.BlockSpec((1, W), lambda i: (0, i)),
                pl.BlockSpec((1, W), lambda i: (0, i)),
            ],
            out_specs=[
                pl.BlockSpec((W, WORD_DIM), lambda i: (i, 0)),
                pl.BlockSpec((W, POS_DIM), lambda i: (i, 0)),
                pl.BlockSpec((W, REL_DIM), lambda i: (i, 0)),
            ],
            core_axis_name=("c", "s"),
            dimension_semantics=(pltpu.PARALLEL,),
        )(wi_hbm, pi_hbm, ri_hbm, wo_hbm, po_hbm, ro_hbm)

    return kern(word_table, pos_table, rel_table,
                word_idxs.reshape(1, N), pos_idxs.reshape(1, N),
                rel_idxs.reshape(1, N))


@jax.jit
def kernel(word_idxs, pos_idxs, rel_idxs, word_table, pos_table, rel_table):
    wo, po, ro = _sc_gather3(word_idxs, pos_idxs, rel_idxs,
                             word_table, pos_table, rel_table)
    return (wo.reshape(B, L, WORD_DIM),
            po.reshape(B, L, POS_DIM),
            ro.reshape(B, L, REL_DIM))


# speedup vs baseline: 1.6499x; 1.6499x over previous
import jax
import jax.numpy as jnp
from jax.experimental import pallas as pl
from jax.experimental.pallas import tpu as pltpu
from jax.experimental.pallas import tpu_sc as plsc

B, L = 4096, 50
N = B * L
WORD_DIM = 64
POS_DIM = 32
REL_DIM = 32
W = 128
GRID = N // W


def _sc_gather3(word_idxs, pos_idxs, rel_idxs, word_table, pos_table, rel_table):
    mesh = plsc.VectorSubcoreMesh(core_axis_name="c", subcore_axis_name="s")

    @pl.kernel(
        out_type=(
            jax.ShapeDtypeStruct((N, WORD_DIM), jnp.float32),
            jax.ShapeDtypeStruct((N, POS_DIM), jnp.float32),
            jax.ShapeDtypeStruct((N, REL_DIM), jnp.float32),
        ),
        mesh=mesh,
        compiler_params=pltpu.CompilerParams(use_tc_tiling_on_sc=False),
    )
    def kern(wt_hbm, pt_hbm, rt_hbm, wi_hbm, pi_hbm, ri_hbm,
             wo_hbm, po_hbm, ro_hbm):
        def body(wi_v, pi_v, ri_v, wo_v, po_v, ro_v):
            pltpu.sync_copy(wt_hbm.at[wi_v.at[0]], wo_v)
            pltpu.sync_copy(pt_hbm.at[pi_v.at[0]], po_v)
            pltpu.sync_copy(rt_hbm.at[ri_v.at[0]], ro_v)

        pltpu.emit_pipeline(
            body,
            grid=(GRID,),
            in_specs=[
                pl.BlockSpec((1, W), lambda i: (0, i)),
                pl.BlockSpec((1, W), lambda i: (0, i)),
                pl.BlockSpec((1, W), lambda i: (0, i)),
            ],
            out_specs=[
                pl.BlockSpec((W, WORD_DIM), lambda i: (i, 0)),
                pl.BlockSpec((W, POS_DIM), lambda i: (i, 0)),
                pl.BlockSpec((W, REL_DIM), lambda i: (i, 0)),
            ],
            core_axis_name=("c", "s"),
            dimension_semantics=(pltpu.PARALLEL,),
        )(wi_hbm, pi_hbm, ri_hbm, wo_hbm, po_hbm, ro_hbm)

    return kern(word_table, pos_table, rel_table,
                word_idxs.reshape(1, N), pos_idxs.reshape(1, N),
                rel_idxs.reshape(1, N))


@jax.jit
def kernel(word_idxs, pos_idxs, rel_idxs, word_table, pos_table, rel_table):
    wo, po, ro = _sc_gather3(word_idxs, pos_idxs, rel_idxs,
                             word_table, pos_table, rel_table)
    return (wo.reshape(B, L, WORD_DIM),
            po.reshape(B, L, POS_DIM),
            ro.reshape(B, L, REL_DIM))
